# baseline (speedup 1.0000x reference)
import functools

import jax
import jax.numpy as jnp
from jax import lax
from jax.experimental import pallas as pl
from jax.experimental.pallas import tpu as pltpu
from jax.experimental.pallas import tpu_sc as plsc

CHUNK = 128
NBUF = 2
LANES = 16


@functools.partial(jax.jit, static_argnums=(2, 3))
def _gather_rows(table2, idx2, n_rows, d):
    info = plsc.get_sparse_core_info()
    nc, ns = info.num_cores, info.num_subcores
    nw = nc * ns
    rows_pw = n_rows // nw
    n_body = rows_pw // NBUF - 1
    d2 = 2 * d
    sr = CHUNK * d // d2

    mesh = plsc.VectorSubcoreMesh(core_axis_name="c", subcore_axis_name="s")

    @functools.partial(
        pl.kernel,
        mesh=mesh,
        compiler_params=pltpu.CompilerParams(use_tc_tiling_on_sc=True, needs_layout_passes=False),
        out_type=jax.ShapeDtypeStruct((n_rows * CHUNK // 2, d2), jnp.float32),
        scratch_types=[
            pltpu.VMEM((rows_pw, CHUNK), jnp.int32),
            pltpu.VMEM((NBUF, CHUNK), jnp.int32),
            pltpu.VMEM((CHUNK, d2), jnp.float32),
            pltpu.VMEM((CHUNK, d2), jnp.float32),
            pltpu.VMEM((sr, d2), jnp.float32),
            pltpu.VMEM((sr, d2), jnp.float32),
            pltpu.SemaphoreType.DMA,
            pltpu.SemaphoreType.DMA,
            pltpu.SemaphoreType.DMA,
            pltpu.SemaphoreType.DMA,
        ],
    )
    def k(tab, idx_hbm, out_hbm, idx_v, pidx, pair0, pair1, st0, st1,
          g0, g1, s0, s1):
        wid = lax.axis_index("s") * nc + lax.axis_index("c")
        base = wid * rows_pw

        pltpu.sync_copy(idx_hbm.at[pl.ds(base, rows_pw)], idx_v)

        iota = lax.iota(jnp.int32, LANES)

        def prep_gather(stage, b, pair_v, sem):
            for g in range(CHUNK // LANES):
                iv = idx_v[stage, pl.ds(LANES * g, LANES)]
                pidx[b, pl.ds(LANES * g, LANES)] = iv >> 1
            pltpu.async_copy(tab.at[pidx.at[b]], pair_v, sem)

        def select(stage, pair_v, st_v):
            UNROLL = 16

            def kbody(kk, carry):
                hk, ck, tvec, rvec = carry
                for _ in range(UNROLL):
                    v = plsc.load_gather(pair_v, [tvec, hk])
                    plsc.store_scatter(st_v, [rvec, ck], v)
                    hk = hk + 1
                    ck = ck + 1
                return (hk, ck, tvec, rvec)

            for g in range(CHUNK // LANES):
                iv = idx_v[stage, pl.ds(LANES * g, LANES)]
                h64 = (iv & 1) * d
                tvec = LANES * g + iota
                rvec = (LANES * g + iota) // 2
                cbase = (iota & 1) * d
                lax.fori_loop(0, d // UNROLL, kbody, (h64, cbase, tvec, rvec))

        def drain(sem, src, dst):
            pltpu.make_async_copy(src, dst, sem).wait()

        def fire_store(stage, st_v, sem):
            pltpu.async_copy(
                st_v, out_hbm.at[pl.ds((base + stage) * sr, sr)], sem)

        prep_gather(0, 0, pair0, g0)
        prep_gather(1, 1, pair1, g1)

        def body(t, carry):
            stg = NBUF * t
            drain(g0, tab.at[pidx.at[0]], pair0)
            select(stg, pair0, st0)
            fire_store(stg, st0, s0)
            drain(s0, st0, out_hbm.at[pl.ds(base * sr, sr)])
            prep_gather(stg + NBUF, 0, pair0, g0)
            drain(g1, tab.at[pidx.at[1]], pair1)
            select(stg + 1, pair1, st1)
            fire_store(stg + 1, st1, s1)
            drain(s1, st1, out_hbm.at[pl.ds(base * sr, sr)])
            prep_gather(stg + 1 + NBUF, 1, pair1, g1)
            return carry

        lax.fori_loop(0, n_body, body, 0)

        last = NBUF * n_body
        drain(g0, tab.at[pidx.at[0]], pair0)
        select(last, pair0, st0)
        fire_store(last, st0, s0)
        drain(g1, tab.at[pidx.at[1]], pair1)
        select(last + 1, pair1, st1)
        fire_store(last + 1, st1, s1)
        drain(s0, st0, out_hbm.at[pl.ds(base * sr, sr)])
        drain(s1, st1, out_hbm.at[pl.ds(base * sr, sr)])

    return k(table2, idx2)


def kernel(token_ids, weight):
    b0, b1 = token_ids.shape
    v, d = weight.shape
    b = b0 * b1
    assert b % CHUNK == 0 and v % 2 == 0
    n_rows = b // CHUNK
    idx2 = token_ids.astype(jnp.int32).reshape(n_rows, CHUNK)
    table2 = weight.reshape(v // 2, 2 * d)
    out = _gather_rows(table2, idx2, n_rows, d)
    return out.reshape(b0, b1, d)


# speedup vs baseline: 2.5045x; 2.5045x over previous
import functools

import jax
import jax.numpy as jnp
from jax import lax
from jax.experimental import pallas as pl
from jax.experimental.pallas import tpu as pltpu
from jax.experimental.pallas import tpu_sc as plsc

CHUNK = 128
K = 4
NBUF = 2


@functools.partial(jax.jit, static_argnums=(2, 3))
def _gather_rows(weight, idx2, n_rows, d):
    info = plsc.get_sparse_core_info()
    nc, ns = info.num_cores, info.num_subcores
    nw = nc * ns
    rows_pw = n_rows // nw
    n_stages = rows_pw // K
    n_body = n_stages // NBUF - 1

    mesh = plsc.VectorSubcoreMesh(core_axis_name="c", subcore_axis_name="s")

    @functools.partial(
        pl.kernel,
        mesh=mesh,
        compiler_params=pltpu.CompilerParams(use_tc_tiling_on_sc=False),
        out_type=jax.ShapeDtypeStruct((n_rows, CHUNK, d), jnp.float32),
        scratch_types=[
            pltpu.VMEM((rows_pw, CHUNK), jnp.int32),
            pltpu.VMEM((K, CHUNK, d), jnp.float32),
            pltpu.VMEM((K, CHUNK, d), jnp.float32),
            pltpu.SemaphoreType.DMA,
            pltpu.SemaphoreType.DMA,
            pltpu.SemaphoreType.DMA,
            pltpu.SemaphoreType.DMA,
        ],
    )
    def k(table_hbm, idx_hbm, out_hbm, idx_v, rows0, rows1, g0, g1, s0, s1):
        wid = lax.axis_index("s") * nc + lax.axis_index("c")
        base = wid * rows_pw

        pltpu.sync_copy(idx_hbm.at[pl.ds(base, rows_pw)], idx_v)

        def fire_gather(stage, rows_v, sem):
            for j in range(K):
                pltpu.async_copy(
                    table_hbm.at[idx_v.at[stage * K + j]], rows_v.at[j], sem
                )

        def drain_gather(rows_v, sem):
            for j in range(K):
                pltpu.make_async_copy(
                    table_hbm.at[idx_v.at[j]], rows_v.at[j], sem
                ).wait()

        def drain_store(rows_v, sem):
            pltpu.make_async_copy(
                rows_v, out_hbm.at[pl.ds(base, K)], sem
            ).wait()

        def fire_store(stage, rows_v, sem):
            pltpu.async_copy(rows_v, out_hbm.at[pl.ds(base + stage * K, K)], sem)

        fire_gather(0, rows0, g0)
        fire_gather(1, rows1, g1)

        def body(t, carry):
            st0 = NBUF * t
            drain_gather(rows0, g0)
            fire_store(st0, rows0, s0)
            drain_store(rows0, s0)
            fire_gather(st0 + NBUF, rows0, g0)
            drain_gather(rows1, g1)
            fire_store(st0 + 1, rows1, s1)
            drain_store(rows1, s1)
            fire_gather(st0 + 1 + NBUF, rows1, g1)
            return carry

        lax.fori_loop(0, n_body, body, 0)

        last0 = NBUF * n_body
        drain_gather(rows0, g0)
        fire_store(last0, rows0, s0)
        drain_gather(rows1, g1)
        fire_store(last0 + 1, rows1, s1)
        drain_store(rows0, s0)
        drain_store(rows1, s1)

    return k(weight, idx2)


def kernel(token_ids, weight):
    b0, b1 = token_ids.shape
    v, d = weight.shape
    b = b0 * b1
    assert b % CHUNK == 0
    n_rows = b // CHUNK
    idx2 = token_ids.astype(jnp.int32).reshape(n_rows, CHUNK)
    out = _gather_rows(weight, idx2, n_rows, d)
    return out.reshape(b0, b1, d)
